# baseline (speedup 1.0000x reference)
import functools

import jax
import jax.numpy as jnp
from jax import lax
from jax.experimental import pallas as pl
from jax.experimental.pallas import tpu as pltpu
from jax.experimental.pallas import tpu_sc as plsc

TABLE_ROWS = 1000
D = 128
B = 16384
NC = 2
NS = 16
NW = NC * NS
B_PER_W = B // NW
CHUNK = 128
N_CHUNK = B_PER_W // CHUNK


def _mlp_body(buf_ref, w1_ref, b1_ref, w2_ref, b2_ref, out_ref):
    h = jnp.dot(buf_ref[...], w1_ref[...], preferred_element_type=jnp.float32)
    h = h + b1_ref[...]
    h = h * jax.nn.sigmoid(h)
    o = jnp.dot(h, w2_ref[...], preferred_element_type=jnp.float32)
    o = o + b2_ref[...]
    out_ref[...] = o * jax.nn.sigmoid(o)


def _mlp_table(buffer, W1, b1, W2, b2):
    return pl.pallas_call(
        _mlp_body,
        out_shape=jax.ShapeDtypeStruct((TABLE_ROWS, D), jnp.float32),
    )(buffer, W1, b1.reshape(1, D), W2, b2.reshape(1, D))


@functools.lru_cache(maxsize=1)
def _make_gather():
    @functools.partial(
        pl.kernel,
        out_type=jax.ShapeDtypeStruct((B, D), jnp.float32),
        scratch_types=[
            pltpu.VMEM((N_CHUNK, CHUNK), jnp.int32),
            pltpu.VMEM((B_PER_W, D), jnp.float32),
        ]
        + [pltpu.SemaphoreType.DMA] * (N_CHUNK + 1),
        mesh=plsc.VectorSubcoreMesh(core_axis_name="c", subcore_axis_name="s"),
    )
    def _gather(steps_hbm, table_hbm, out_hbm, idx_v, rows_v, *sems):
        gsems, wsem = sems[:N_CHUNK], sems[N_CHUNK]
        wid = lax.axis_index("s") * NC + lax.axis_index("c")
        pltpu.sync_copy(steps_hbm.at[pl.ds(wid * N_CHUNK, N_CHUNK)], idx_v)
        gathers = [
            pltpu.async_copy(
                table_hbm.at[idx_v.at[j]],
                rows_v.at[pl.ds(j * CHUNK, CHUNK)],
                gsems[j],
            )
            for j in range(N_CHUNK)
        ]
        writes = []
        for j in range(N_CHUNK):
            gathers[j].wait()
            writes.append(
                pltpu.async_copy(
                    rows_v.at[pl.ds(j * CHUNK, CHUNK)],
                    out_hbm.at[pl.ds(wid * B_PER_W + j * CHUNK, CHUNK)],
                    wsem,
                )
            )
        for w in writes:
            w.wait()

    return _gather


def kernel(steps, buffer, W1, b1, W2, b2):
    table = _mlp_table(buffer, W1, b1, W2, b2)
    steps2 = steps.astype(jnp.int32).reshape(B // CHUNK, CHUNK)
    return _make_gather()(steps2, table)


# speedup vs baseline: 1.1534x; 1.1534x over previous
import functools

import jax
import jax.numpy as jnp
from jax import lax
from jax.experimental import pallas as pl
from jax.experimental.pallas import tpu as pltpu
from jax.experimental.pallas import tpu_sc as plsc

TABLE_ROWS = 1000
D = 128
B = 16384
NC = 2
NS = 16
NW = NC * NS
B_PER_W = B // NW
CHUNK = 128
N_CHUNK = B_PER_W // CHUNK


def _mlp_body(buf_ref, w1_ref, b1_ref, w2_ref, b2_ref, out_ref):
    h = jnp.dot(buf_ref[...], w1_ref[...], preferred_element_type=jnp.float32)
    h = h + b1_ref[...]
    h = h * jax.nn.sigmoid(h)
    o = jnp.dot(h, w2_ref[...], preferred_element_type=jnp.float32)
    o = o + b2_ref[...]
    out_ref[...] = o * jax.nn.sigmoid(o)


def _mlp_table(buffer, W1, b1, W2, b2):
    return pl.pallas_call(
        _mlp_body,
        out_shape=jax.ShapeDtypeStruct((TABLE_ROWS, D), jnp.float32),
    )(buffer, W1, b1.reshape(1, D), W2, b2.reshape(1, D))


@functools.lru_cache(maxsize=1)
def _make_gather():
    @functools.partial(
        pl.kernel,
        out_type=jax.ShapeDtypeStruct((B, D), jnp.float32),
        scratch_types=[
            pltpu.VMEM((N_CHUNK, CHUNK), jnp.int32),
            pltpu.VMEM((B_PER_W, D), jnp.float32),
            pltpu.VMEM_SHARED((TABLE_ROWS, D), jnp.float32),
        ]
        + [pltpu.SemaphoreType.DMA] * (N_CHUNK + 1),
        mesh=plsc.VectorSubcoreMesh(core_axis_name="c", subcore_axis_name="s"),
    )
    def _gather(steps_hbm, table_hbm, out_hbm, idx_v, rows_v, table_s, *sems):
        gsems, wsem = sems[:N_CHUNK], sems[N_CHUNK]
        sid = lax.axis_index("s")
        wid = sid * NC + lax.axis_index("c")

        @pl.when(sid == 0)
        def _stage_table():
            pltpu.sync_copy(table_hbm, table_s)

        pltpu.sync_copy(steps_hbm.at[pl.ds(wid * N_CHUNK, N_CHUNK)], idx_v)
        plsc.subcore_barrier()
        gathers = [
            pltpu.async_copy(
                table_s.at[idx_v.at[j]],
                rows_v.at[pl.ds(j * CHUNK, CHUNK)],
                gsems[j],
            )
            for j in range(N_CHUNK)
        ]
        writes = []
        for j in range(N_CHUNK):
            gathers[j].wait()
            writes.append(
                pltpu.async_copy(
                    rows_v.at[pl.ds(j * CHUNK, CHUNK)],
                    out_hbm.at[pl.ds(wid * B_PER_W + j * CHUNK, CHUNK)],
                    wsem,
                )
            )
        for w in writes:
            w.wait()

    return _gather


def kernel(steps, buffer, W1, b1, W2, b2):
    table = _mlp_table(buffer, W1, b1, W2, b2)
    steps2 = steps.astype(jnp.int32).reshape(B // CHUNK, CHUNK)
    return _make_gather()(steps2, table)


# speedup vs baseline: 1.1795x; 1.0226x over previous
import functools

import jax
import jax.numpy as jnp
from jax import lax
from jax.experimental import pallas as pl
from jax.experimental.pallas import tpu as pltpu
from jax.experimental.pallas import tpu_sc as plsc

TABLE_ROWS = 1000
TABLE_PAD = 1024
D = 128
B = 16384
NC = 2
NS = 16
NW = NC * NS
B_PER_W = B // NW
CHUNK = 128
N_CHUNK = B_PER_W // CHUNK


def _mlp_body(buf_ref, w1_ref, b1_ref, w2_ref, b2_ref, out_ref):
    h = jnp.dot(buf_ref[...], w1_ref[...], preferred_element_type=jnp.float32)
    h = h + b1_ref[...]
    h = h * jax.nn.sigmoid(h)
    o = jnp.dot(h, w2_ref[...], preferred_element_type=jnp.float32)
    o = o + b2_ref[...]
    out_ref[0:TABLE_ROWS, :] = o * jax.nn.sigmoid(o)
    out_ref[TABLE_ROWS:TABLE_PAD, :] = jnp.zeros(
        (TABLE_PAD - TABLE_ROWS, D), jnp.float32
    )


def _mlp_table(buffer, W1, b1, W2, b2):
    return pl.pallas_call(
        _mlp_body,
        out_shape=jax.ShapeDtypeStruct((TABLE_PAD, D), jnp.float32),
    )(buffer, W1, b1.reshape(1, D), W2, b2.reshape(1, D))


@functools.lru_cache(maxsize=1)
def _make_gather():
    @functools.partial(
        pl.kernel,
        out_type=jax.ShapeDtypeStruct((B, D), jnp.float32),
        scratch_types=[
            pltpu.VMEM((N_CHUNK, CHUNK), jnp.int32),
            pltpu.VMEM((B_PER_W, D), jnp.float32),
            pltpu.VMEM_SHARED((TABLE_PAD, D), jnp.float32),
        ]
        + [pltpu.SemaphoreType.DMA] * (N_CHUNK + 2),
        mesh=plsc.VectorSubcoreMesh(core_axis_name="c", subcore_axis_name="s"),
    )
    def _gather(steps_hbm, table_hbm, out_hbm, idx_v, rows_v, table_s, *sems):
        gsems, wsem, tsem = sems[:N_CHUNK], sems[N_CHUNK], sems[N_CHUNK + 1]
        sid = lax.axis_index("s")
        wid = sid * NC + lax.axis_index("c")
        rows_per_tile = TABLE_PAD // NS
        stage = pltpu.async_copy(
            table_hbm.at[pl.ds(sid * rows_per_tile, rows_per_tile)],
            table_s.at[pl.ds(sid * rows_per_tile, rows_per_tile)],
            tsem,
        )
        pltpu.sync_copy(steps_hbm.at[pl.ds(wid * N_CHUNK, N_CHUNK)], idx_v)
        stage.wait()
        plsc.subcore_barrier()
        gathers = [
            pltpu.async_copy(
                table_s.at[idx_v.at[j]],
                rows_v.at[pl.ds(j * CHUNK, CHUNK)],
                gsems[j],
            )
            for j in range(N_CHUNK)
        ]
        writes = []
        for j in range(N_CHUNK):
            gathers[j].wait()
            writes.append(
                pltpu.async_copy(
                    rows_v.at[pl.ds(j * CHUNK, CHUNK)],
                    out_hbm.at[pl.ds(wid * B_PER_W + j * CHUNK, CHUNK)],
                    wsem,
                )
            )
        for w in writes:
            w.wait()

    return _gather


def kernel(steps, buffer, W1, b1, W2, b2):
    table = _mlp_table(buffer, W1, b1, W2, b2)
    steps2 = steps.astype(jnp.int32).reshape(B // CHUNK, CHUNK)
    return _make_gather()(steps2, table)
